# Initial kernel scaffold; baseline (speedup 1.0000x reference)
import jax
import jax.numpy as jnp
from jax.experimental import pallas as pl


def kernel(questions, category, question_table, category_table, pos_table):
    raise NotImplementedError("write your pallas kernel here")


# speedup vs baseline: 2.9581x; 2.9581x over previous
import jax
import jax.numpy as jnp
from jax import lax
from jax.experimental import pallas as pl
from jax.experimental.pallas import tpu as pltpu
from jax.experimental.pallas import tpu_sc as plsc

B = 1024
SEQ = 200
EMB = 64
NC = 2
NS = 16
NW = NC * NS
ITEMS = B * SEQ
IPT = ITEMS // NW
CHUNK = 40
NCHUNK = IPT // CHUNK
LANES = 16


def _body(q_idx_hbm, c_idx_hbm, qtab_hbm, ctab_hbm, pos_hbm, out_hbm,
          q_idx_v, c_idx_v, pos_v, q_buf, c_buf, sem_g):
    wid = lax.axis_index("s") * NC + lax.axis_index("c")
    base = wid * IPT
    pltpu.sync_copy(q_idx_hbm.at[pl.ds(base, IPT)], q_idx_v)
    pltpu.sync_copy(c_idx_hbm.at[pl.ds(base, IPT)], c_idx_v)
    pltpu.sync_copy(pos_hbm, pos_v)

    @pl.loop(0, NCHUNK)
    def _chunk(k):
        off = k * CHUNK
        s0 = lax.rem(off, SEQ)
        cq = pltpu.async_copy(qtab_hbm.at[q_idx_v.at[pl.ds(off, CHUNK)]],
                              q_buf, sem_g)
        cc = pltpu.async_copy(ctab_hbm.at[c_idx_v.at[pl.ds(off, CHUNK)]],
                              c_buf, sem_g)
        cq.wait()
        cc.wait()

        @pl.loop(0, CHUNK)
        def _item(i):
            row = s0 + i
            for c in range(EMB // LANES):
                col = c * LANES
                qv = q_buf[i, pl.ds(col, LANES)]
                pv = pos_v[row, pl.ds(col, LANES)]
                plsc.addupdate(c_buf.at[i, pl.ds(col, LANES)], qv + pv)

        pltpu.sync_copy(c_buf, out_hbm.at[pl.ds(base + off, CHUNK)])


def kernel(questions, category, question_table, category_table, pos_table):
    q = questions.reshape(ITEMS).astype(jnp.int32)
    c = category.reshape(ITEMS).astype(jnp.int32)
    out = pl.kernel(
        _body,
        out_type=jax.ShapeDtypeStruct((ITEMS, EMB), jnp.float32),
        mesh=plsc.VectorSubcoreMesh(core_axis_name="c", subcore_axis_name="s"),
        compiler_params=pltpu.CompilerParams(use_tc_tiling_on_sc=False),
        scratch_types=[
            pltpu.VMEM((IPT,), jnp.int32),
            pltpu.VMEM((IPT,), jnp.int32),
            pltpu.VMEM((SEQ, EMB), jnp.float32),
            pltpu.VMEM((CHUNK, EMB), jnp.float32),
            pltpu.VMEM((CHUNK, EMB), jnp.float32),
            pltpu.SemaphoreType.DMA,
        ],
    )(q, c, question_table, category_table, pos_table)
    return out.reshape(B, SEQ, EMB)


# speedup vs baseline: 4.9302x; 1.6667x over previous
import jax
import jax.numpy as jnp
from jax import lax
from jax.experimental import pallas as pl
from jax.experimental.pallas import tpu as pltpu
from jax.experimental.pallas import tpu_sc as plsc

B = 1024
SEQ = 200
EMB = 64
NC = 2
NS = 16
NW = NC * NS
ITEMS = B * SEQ
IPT = ITEMS // NW
CHUNK = SEQ
HALF = SEQ // 2
NCHUNK = IPT // CHUNK
NBUF = 4
PRE = 2
LANES = 16


def _fire_chunk(qtab_hbm, ctab_hbm, q_idx_v, c_idx_v, q_buf, o_buf, gsem, j, b):
    for h in range(2):
        dst = o_buf.at[b, pl.ds(h * HALF, HALF)]
        pltpu.async_copy(ctab_hbm.at[c_idx_v.at[2 * j + h]], dst, gsem[b])
        dst = q_buf.at[b, pl.ds(h * HALF, HALF)]
        pltpu.async_copy(qtab_hbm.at[q_idx_v.at[2 * j + h]], dst, gsem[b])


def _wait_chunk(qtab_hbm, ctab_hbm, q_idx_v, c_idx_v, q_buf, o_buf, gsem, j, b):
    for h in range(2):
        dst = o_buf.at[b, pl.ds(h * HALF, HALF)]
        pltpu.make_async_copy(ctab_hbm.at[c_idx_v.at[2 * j + h]], dst, gsem[b]).wait()
        dst = q_buf.at[b, pl.ds(h * HALF, HALF)]
        pltpu.make_async_copy(qtab_hbm.at[q_idx_v.at[2 * j + h]], dst, gsem[b]).wait()


def _body(q_idx_hbm, c_idx_hbm, qtab_hbm, ctab_hbm, pos_hbm, out_hbm,
          q_idx_v, c_idx_v, pos_v, q_buf, o_buf, gsem, osem):
    wid = lax.axis_index("s") * NC + lax.axis_index("c")
    base = wid * IPT
    row0 = wid * (2 * NCHUNK)
    pltpu.sync_copy(q_idx_hbm.at[pl.ds(row0, 2 * NCHUNK)], q_idx_v)
    pltpu.sync_copy(c_idx_hbm.at[pl.ds(row0, 2 * NCHUNK)], c_idx_v)
    pltpu.sync_copy(pos_hbm, pos_v)

    for j in range(PRE):
        _fire_chunk(qtab_hbm, ctab_hbm, q_idx_v, c_idx_v, q_buf, o_buf,
                    gsem, j, j % NBUF)

    @pl.loop(0, NCHUNK, step=NBUF)
    def _turns(k):
        for b in range(NBUF):
            cur = k + b
            _wait_chunk(qtab_hbm, ctab_hbm, q_idx_v, c_idx_v, q_buf, o_buf,
                        gsem, cur, b)

            @pl.loop(0, CHUNK, unroll=2)
            def _item(i):
                for c in range(EMB // LANES):
                    col = c * LANES
                    qv = q_buf[b, i, pl.ds(col, LANES)]
                    pv = pos_v[i, pl.ds(col, LANES)]
                    plsc.addupdate(o_buf.at[b, i, pl.ds(col, LANES)], qv + pv)

            out_dst = out_hbm.at[pl.ds(base + cur * CHUNK, CHUNK)]
            pltpu.async_copy(o_buf.at[b], out_dst, osem[b])

            nxt = cur + PRE
            bn = (b + PRE) % NBUF

            @pl.when(nxt < NCHUNK)
            def _():
                @pl.when(cur >= NBUF - PRE)
                def _():
                    prev = nxt - NBUF
                    src = o_buf.at[bn]
                    dst = out_hbm.at[pl.ds(base + prev * CHUNK, CHUNK)]
                    pltpu.make_async_copy(src, dst, osem[bn]).wait()
                _fire_chunk(qtab_hbm, ctab_hbm, q_idx_v, c_idx_v, q_buf,
                            o_buf, gsem, nxt, bn)

    for j in range(NCHUNK - NBUF, NCHUNK):
        b = j % NBUF
        src = o_buf.at[b]
        dst = out_hbm.at[pl.ds(base + j * CHUNK, CHUNK)]
        pltpu.make_async_copy(src, dst, osem[b]).wait()


def kernel(questions, category, question_table, category_table, pos_table):
    q = questions.reshape(ITEMS // HALF, HALF).astype(jnp.int32)
    c = category.reshape(ITEMS // HALF, HALF).astype(jnp.int32)
    out = pl.kernel(
        _body,
        out_type=jax.ShapeDtypeStruct((ITEMS, EMB), jnp.float32),
        mesh=plsc.VectorSubcoreMesh(core_axis_name="c", subcore_axis_name="s"),
        compiler_params=pltpu.CompilerParams(use_tc_tiling_on_sc=False),
        scratch_types=[
            pltpu.VMEM((2 * NCHUNK, HALF), jnp.int32),
            pltpu.VMEM((2 * NCHUNK, HALF), jnp.int32),
            pltpu.VMEM((SEQ, EMB), jnp.float32),
            pltpu.VMEM((NBUF, CHUNK, EMB), jnp.float32),
            pltpu.VMEM((NBUF, CHUNK, EMB), jnp.float32),
            [pltpu.SemaphoreType.DMA] * NBUF,
            [pltpu.SemaphoreType.DMA] * NBUF,
        ],
    )(q, c, question_table, category_table, pos_table)
    return out.reshape(B, SEQ, EMB)
